# Initial kernel scaffold; baseline (speedup 1.0000x reference)
import jax
import jax.numpy as jnp
from jax.experimental import pallas as pl


def kernel(prices, volumes, holding_times, unique_addresses, transaction_counts, contract_calls, W_benford, b_benford, W_round, b_round, W_turn, b_turn, W_act, b_act):
    raise NotImplementedError("write your pallas kernel here")


# speedup vs baseline: 1.2427x; 1.2427x over previous
import math

import jax
import jax.numpy as jnp
from jax.experimental import pallas as pl
from jax.experimental.pallas import tpu as pltpu

_L = 200
_B_BLK = 512


def _body(prices_ref, vol_ref, hold_ref, scal_ref, w_ref, bias_ref, out_ref):
    linv = 1.0 / _L

    p = prices_ref[...]
    pi = jnp.abs(p).astype(jnp.int32)
    pif = pi.astype(jnp.float32)
    nf = jnp.where(pi == 0, 1.0, pif)

    p10 = jnp.ones_like(nf)
    t = 10.0
    for _ in range(9):
        p10 = jnp.where(nf >= t, t, p10)
        t = t * 10.0
    fd = jnp.ones_like(nf)
    for k in range(2, 10):
        fd = fd + (nf >= float(k) * p10).astype(jnp.float32)

    q = jnp.floor(pif / 10.0)
    last = pif - q * 10.0

    feats = []
    for k in range(1, 10):
        ck = jnp.sum((fd == float(k)).astype(jnp.float32), axis=1, keepdims=True)
        ek = math.log10((k + 1.0) / float(k))
        feats.append(jnp.abs(ck * linv - ek))
    for k in range(10):
        rk = jnp.sum((last == float(k)).astype(jnp.float32), axis=1, keepdims=True)
        feats.append(rk * linv)

    h = hold_ref[...]
    hm = jnp.sum(h, axis=1, keepdims=True) * linv
    hdev = h - hm
    hstd = jnp.sqrt(jnp.sum(hdev * hdev, axis=1, keepdims=True) * (1.0 / (_L - 1)))
    feats.append(hm)
    feats.append(jnp.min(h, axis=1, keepdims=True))
    feats.append(jnp.max(h, axis=1, keepdims=True))
    feats.append(hstd)

    v = vol_ref[...]
    vsum = jnp.sum(v, axis=1, keepdims=True)
    vm = vsum * linv
    vdev = v - vm
    vstd = jnp.sqrt(jnp.sum(vdev * vdev, axis=1, keepdims=True) * (1.0 / (_L - 1)))
    feats.append(scal_ref[:, 0:1])
    feats.append(scal_ref[:, 1:2])
    feats.append(scal_ref[:, 2:3])
    feats.append(vm)
    feats.append(vstd)
    feats.append(vsum)

    acc = jnp.zeros((_B_BLK, 32), dtype=jnp.float32)
    for j, f in enumerate(feats):
        acc = acc + f * w_ref[j : j + 1, :]
    out_ref[...] = acc + bias_ref[...]


def kernel(prices, volumes, holding_times, unique_addresses, transaction_counts,
           contract_calls, W_benford, b_benford, W_round, b_round,
           W_turn, b_turn, W_act, b_act):
    B = prices.shape[0]
    scal = jnp.stack([unique_addresses, transaction_counts, contract_calls], axis=-1)
    wall = jax.scipy.linalg.block_diag(W_benford.T, W_round.T, W_turn.T, W_act.T)
    wall = jnp.pad(wall, ((0, 32 - wall.shape[0]), (0, 0)))
    bias = jnp.concatenate([b_benford, b_round, b_turn, b_act]).reshape(1, 32)

    grid = (B // _B_BLK,)
    return pl.pallas_call(
        _body,
        grid=grid,
        in_specs=[
            pl.BlockSpec((_B_BLK, _L), lambda i: (i, 0)),
            pl.BlockSpec((_B_BLK, _L), lambda i: (i, 0)),
            pl.BlockSpec((_B_BLK, _L), lambda i: (i, 0)),
            pl.BlockSpec((_B_BLK, 3), lambda i: (i, 0)),
            pl.BlockSpec((32, 32), lambda i: (0, 0)),
            pl.BlockSpec((1, 32), lambda i: (0, 0)),
        ],
        out_specs=pl.BlockSpec((_B_BLK, 32), lambda i: (i, 0)),
        out_shape=jax.ShapeDtypeStruct((B, 32), jnp.float32),
        compiler_params=pltpu.CompilerParams(
            dimension_semantics=("parallel",),
        ),
    )(prices, volumes, holding_times, scal, wall, bias)


# speedup vs baseline: 1.2813x; 1.0311x over previous
import math

import jax
import jax.numpy as jnp
from jax.experimental import pallas as pl
from jax.experimental.pallas import tpu as pltpu

_L = 200
_B_BLK = 512


def _counts_fast(pif):
    cs = [jnp.sum((pif == float(k)).astype(jnp.float32), axis=1, keepdims=True)
          for k in range(10)]
    ben = [cs[0] + cs[1]] + cs[2:]
    return tuple(ben + cs)


def _counts_general(pif):
    nf = jnp.maximum(pif, 1.0)
    p10 = jnp.ones_like(nf)
    t = 10.0
    for _ in range(9):
        p10 = jnp.where(nf >= t, t, p10)
        t = t * 10.0
    fd = jnp.ones_like(nf)
    for k in range(2, 10):
        fd = fd + (nf >= float(k) * p10).astype(jnp.float32)
    last = pif - jnp.floor(pif / 10.0) * 10.0
    ben = [jnp.sum((fd == float(k)).astype(jnp.float32), axis=1, keepdims=True)
           for k in range(1, 10)]
    rnd = [jnp.sum((last == float(k)).astype(jnp.float32), axis=1, keepdims=True)
           for k in range(10)]
    return tuple(ben + rnd)


def _body(prices_ref, vol_ref, hold_ref, scal_ref, w_ref, bias_ref, out_ref):
    linv = 1.0 / _L
    dinv = 1.0 / (_L - 1)

    p = prices_ref[...]
    pif = jnp.floor(jnp.abs(p))
    counts = jax.lax.cond(jnp.max(pif) <= 9.0, _counts_fast, _counts_general, pif)

    feats = []
    for k in range(1, 10):
        ek = math.log10((k + 1.0) / float(k))
        feats.append(jnp.abs(counts[k - 1] * linv - ek))
    for k in range(10):
        feats.append(counts[9 + k] * linv)

    h = hold_ref[...]
    hsum = jnp.sum(h, axis=1, keepdims=True)
    hsq = jnp.sum(h * h, axis=1, keepdims=True)
    hm = hsum * linv
    hvar = jnp.maximum(hsq - hsum * hm, 0.0) * dinv
    feats.append(hm)
    feats.append(jnp.min(h, axis=1, keepdims=True))
    feats.append(jnp.max(h, axis=1, keepdims=True))
    feats.append(jnp.sqrt(hvar))

    v = vol_ref[...]
    vsum = jnp.sum(v, axis=1, keepdims=True)
    vsq = jnp.sum(v * v, axis=1, keepdims=True)
    vm = vsum * linv
    vvar = jnp.maximum(vsq - vsum * vm, 0.0) * dinv
    feats.append(scal_ref[:, 0:1])
    feats.append(scal_ref[:, 1:2])
    feats.append(scal_ref[:, 2:3])
    feats.append(vm)
    feats.append(jnp.sqrt(vvar))
    feats.append(vsum)

    acc = jnp.zeros((_B_BLK, 32), dtype=jnp.float32)
    for j, f in enumerate(feats):
        acc = acc + f * w_ref[j : j + 1, :]
    out_ref[...] = acc + bias_ref[...]


def kernel(prices, volumes, holding_times, unique_addresses, transaction_counts,
           contract_calls, W_benford, b_benford, W_round, b_round,
           W_turn, b_turn, W_act, b_act):
    B = prices.shape[0]
    scal = jnp.stack([unique_addresses, transaction_counts, contract_calls], axis=-1)
    wall = jax.scipy.linalg.block_diag(W_benford.T, W_round.T, W_turn.T, W_act.T)
    wall = jnp.pad(wall, ((0, 32 - wall.shape[0]), (0, 0)))
    bias = jnp.concatenate([b_benford, b_round, b_turn, b_act]).reshape(1, 32)

    grid = (B // _B_BLK,)
    return pl.pallas_call(
        _body,
        grid=grid,
        in_specs=[
            pl.BlockSpec((_B_BLK, _L), lambda i: (i, 0)),
            pl.BlockSpec((_B_BLK, _L), lambda i: (i, 0)),
            pl.BlockSpec((_B_BLK, _L), lambda i: (i, 0)),
            pl.BlockSpec((_B_BLK, 3), lambda i: (i, 0)),
            pl.BlockSpec((32, 32), lambda i: (0, 0)),
            pl.BlockSpec((1, 32), lambda i: (0, 0)),
        ],
        out_specs=pl.BlockSpec((_B_BLK, 32), lambda i: (i, 0)),
        out_shape=jax.ShapeDtypeStruct((B, 32), jnp.float32),
        compiler_params=pltpu.CompilerParams(
            dimension_semantics=("parallel",),
        ),
    )(prices, volumes, holding_times, scal, wall, bias)
